# baseline (speedup 1.0000x reference)
import jax
import jax.numpy as jnp
from jax.experimental import pallas as pl
from jax.experimental.pallas import tpu as pltpu

_PENALTY = 0.1


def _controls_body(w_ref, b_ref, obs_ref, s_ref, sm_ref):
    w = w_ref[0, :]
    b = b_ref[0, 0]
    obs = obs_ref[0]
    x = jnp.sum(obs * w[None, None, :], axis=-1) + b
    s = jax.nn.log_sigmoid(x)
    s_ref[0] = s
    sm_ref[0] = s - x


def _gamma_body(scores_ref, gamma_ref, read_ref, write_ref):
    sc = scores_ref[0]
    Tt, Ts = sc.shape
    lane_i = jax.lax.broadcasted_iota(jnp.int32, (Tt, Ts), 1)
    lane_f = lane_i.astype(jnp.float32)
    scp = sc - _PENALTY * (lane_f / Ts)
    m = jnp.max(scp, axis=1, keepdims=True)
    cand = jnp.where(scp == m, lane_i, Ts)
    bc = jnp.min(cand, axis=1, keepdims=True)
    sub_i = jax.lax.broadcasted_iota(jnp.int32, (Tt, 1), 0)
    k = 1
    while k < Tt:
        shifted = pltpu.roll(bc, k, axis=0)
        bc = jnp.maximum(bc, jnp.where(sub_i >= k, shifted, -1))
        k *= 2
    gamma = (lane_i >= bc).astype(jnp.float32)
    gamma_ref[0] = gamma
    write_ref[0] = gamma[1:, :]
    read_ref[0] = 1.0 - gamma[1:, :]


def _run(observations, scores, gate_w, gate_b, interpret=False):
    B, Tt, Ts, C = observations.shape
    Tblk = 32
    s_arr, sm_arr = pl.pallas_call(
        _controls_body,
        grid=(B, Tt // Tblk),
        in_specs=[
            pl.BlockSpec((1, C), lambda b, t: (0, 0)),
            pl.BlockSpec((1, 1), lambda b, t: (0, 0)),
            pl.BlockSpec((1, Tblk, Ts, C), lambda b, t: (b, t, 0, 0)),
        ],
        out_specs=[
            pl.BlockSpec((1, Tblk, Ts), lambda b, t: (b, t, 0)),
            pl.BlockSpec((1, Tblk, Ts), lambda b, t: (b, t, 0)),
        ],
        out_shape=[
            jax.ShapeDtypeStruct((B, Tt, Ts), jnp.float32),
            jax.ShapeDtypeStruct((B, Tt, Ts), jnp.float32),
        ],
        interpret=interpret,
    )(gate_w, gate_b.reshape(1, 1), observations)

    gamma, read, write = pl.pallas_call(
        _gamma_body,
        grid=(B,),
        in_specs=[pl.BlockSpec((1, Tt, Ts), lambda b: (b, 0, 0))],
        out_specs=[
            pl.BlockSpec((1, Tt, Ts), lambda b: (b, 0, 0)),
            pl.BlockSpec((1, Tt - 1, Ts), lambda b: (b, 0, 0)),
            pl.BlockSpec((1, Tt - 1, Ts), lambda b: (b, 0, 0)),
        ],
        out_shape=[
            jax.ShapeDtypeStruct((B, Tt, Ts), jnp.float32),
            jax.ShapeDtypeStruct((B, Tt - 1, Ts), jnp.float32),
            jax.ShapeDtypeStruct((B, Tt - 1, Ts), jnp.float32),
        ],
        interpret=interpret,
    )(scores)

    controls = jnp.stack([s_arr, sm_arr], axis=-1)[:, :-1]
    return controls, gamma, read, write


@jax.jit
def kernel(observations, scores, gate_w, gate_b):
    return _run(observations, scores, gate_w, gate_b)


# speedup vs baseline: 2.1336x; 2.1336x over previous
import jax
import jax.numpy as jnp
from jax.experimental import pallas as pl
from jax.experimental.pallas import tpu as pltpu

_PENALTY = 0.1


def _controls_body(w_ref, b_ref, obs_ref, s_ref, sm_ref):
    b = b_ref[0, 0]
    blk = obs_ref.shape[1]
    ts = obs_ref.shape[2]
    c = obs_ref.shape[3]
    m = obs_ref[0].reshape(blk * ts, c)
    x = jax.lax.dot_general(
        w_ref[...], m, (((1,), (1,)), ((), ())),
        preferred_element_type=jnp.float32,
    ) + b
    s = jax.nn.log_sigmoid(x)
    s_ref[0, 0] = s
    sm_ref[0, 0] = s - x


def _gamma_body(scores_ref, gamma_ref, read_ref, write_ref):
    sc = scores_ref[0]
    Tt, Ts = sc.shape
    lane_i = jax.lax.broadcasted_iota(jnp.int32, (Tt, Ts), 1)
    lane_f = lane_i.astype(jnp.float32)
    scp = sc - _PENALTY * (lane_f / Ts)
    m = jnp.max(scp, axis=1, keepdims=True)
    cand = jnp.where(scp == m, lane_i, Ts)
    bc = jnp.min(cand, axis=1, keepdims=True)
    sub_i = jax.lax.broadcasted_iota(jnp.int32, (Tt, 1), 0)
    k = 1
    while k < Tt:
        shifted = pltpu.roll(bc, k, axis=0)
        bc = jnp.maximum(bc, jnp.where(sub_i >= k, shifted, -1))
        k *= 2
    gamma = (lane_i >= bc).astype(jnp.float32)
    gamma_ref[0] = gamma
    write_ref[0] = gamma[1:, :]
    read_ref[0] = 1.0 - gamma[1:, :]


def _run(observations, scores, gate_w, gate_b, interpret=False):
    B, Tt, Ts, C = observations.shape
    Tblk = 32
    nT = Tt // Tblk
    s_arr, sm_arr = pl.pallas_call(
        _controls_body,
        grid=(B, nT),
        in_specs=[
            pl.BlockSpec((1, C), lambda b, t: (0, 0)),
            pl.BlockSpec((1, 1), lambda b, t: (0, 0)),
            pl.BlockSpec((1, Tblk, Ts, C), lambda b, t: (b, t, 0, 0)),
        ],
        out_specs=[
            pl.BlockSpec((1, 1, 1, Tblk * Ts), lambda b, t: (b, t, 0, 0)),
            pl.BlockSpec((1, 1, 1, Tblk * Ts), lambda b, t: (b, t, 0, 0)),
        ],
        out_shape=[
            jax.ShapeDtypeStruct((B, nT, 1, Tblk * Ts), jnp.float32),
            jax.ShapeDtypeStruct((B, nT, 1, Tblk * Ts), jnp.float32),
        ],
        interpret=interpret,
    )(gate_w, gate_b.reshape(1, 1), observations)
    s_arr = s_arr.reshape(B, Tt, Ts)
    sm_arr = sm_arr.reshape(B, Tt, Ts)

    gamma, read, write = pl.pallas_call(
        _gamma_body,
        grid=(B,),
        in_specs=[pl.BlockSpec((1, Tt, Ts), lambda b: (b, 0, 0))],
        out_specs=[
            pl.BlockSpec((1, Tt, Ts), lambda b: (b, 0, 0)),
            pl.BlockSpec((1, Tt - 1, Ts), lambda b: (b, 0, 0)),
            pl.BlockSpec((1, Tt - 1, Ts), lambda b: (b, 0, 0)),
        ],
        out_shape=[
            jax.ShapeDtypeStruct((B, Tt, Ts), jnp.float32),
            jax.ShapeDtypeStruct((B, Tt - 1, Ts), jnp.float32),
            jax.ShapeDtypeStruct((B, Tt - 1, Ts), jnp.float32),
        ],
        interpret=interpret,
    )(scores)

    controls = jnp.stack([s_arr, sm_arr], axis=-1)[:, :-1]
    return controls, gamma, read, write


@jax.jit
def kernel(observations, scores, gate_w, gate_b):
    return _run(observations, scores, gate_w, gate_b)
